# baseline (speedup 1.0000x reference)
import functools

import jax
import jax.numpy as jnp
from jax import lax
from jax.experimental import pallas as pl
from jax.experimental.pallas import tpu as pltpu
from jax.experimental.pallas import tpu_sc as plsc

_NC = 2
_NS = 16
_L = 16
_NW = _NC * _NS


def _vbcast(vec, h):
    dn = lax.GatherDimensionNumbers(
        offset_dims=(), collapsed_slice_dims=(0,), start_index_map=(0,))
    idx = jnp.full((_L, 1), h, jnp.int32)
    return lax.gather(vec, idx, dn, slice_sizes=(1,),
                      mode=lax.GatherScatterMode.PROMISE_IN_BOUNDS)


def _tc_prologue(stim, W1, b1, Wfc, bfc, *, interpret=False):
    B, H = stim.shape
    BLK = 512
    wfc_row = Wfc.reshape(1, H)
    b1_row = b1.reshape(1, H)
    bfc_11 = bfc.reshape(1, 1)

    def body(stim_ref, w1_ref, wfc_ref, b1_ref, bfc_ref, v_ref, c_ref):
        s = stim_ref[...] * wfc_ref[...]
        v_ref[...] = lax.dot_general(
            s, w1_ref[...], (((1,), (1,)), ((), ())),
            precision=lax.Precision.HIGHEST,
            preferred_element_type=jnp.float32)
        c_ref[...] = jnp.sum(s * b1_ref[...], axis=1, keepdims=True) + bfc_ref[...]

    grid = (B // BLK,)
    v, c = pl.pallas_call(
        body,
        grid=grid,
        in_specs=[
            pl.BlockSpec((BLK, H), lambda i: (i, 0)),
            pl.BlockSpec((H, H), lambda i: (0, 0)),
            pl.BlockSpec((1, H), lambda i: (0, 0)),
            pl.BlockSpec((1, H), lambda i: (0, 0)),
            pl.BlockSpec((1, 1), lambda i: (0, 0)),
        ],
        out_specs=[
            pl.BlockSpec((BLK, H), lambda i: (i, 0)),
            pl.BlockSpec((BLK, 1), lambda i: (i, 0)),
        ],
        out_shape=[
            jax.ShapeDtypeStruct((B, H), jnp.float32),
            jax.ShapeDtypeStruct((B, 1), jnp.float32),
        ],
        interpret=interpret,
    )(stim, W1, wfc_row, b1_row, bfc_11)
    return v, c


def _sc_main(embed, atn, v, c, *, interpret=False):
    B, N = atn.shape
    H = embed.shape[1]
    BPW = B // _NW
    NPAD = 64
    NCH = NPAD // _L
    HCH = H // _L

    mesh = plsc.VectorSubcoreMesh(core_axis_name="c", subcore_axis_name="s",
                                  num_cores=_NC, num_subcores=_NS)

    @functools.partial(
        pl.kernel,
        mesh=mesh,
        out_type=(
            jax.ShapeDtypeStruct((B, NPAD), jnp.float32),
            jax.ShapeDtypeStruct((B,), jnp.int32),
        ),
        scratch_types=[
            pltpu.VMEM((BPW, N), jnp.int32),
            pltpu.VMEM((BPW, H), jnp.float32),
            pltpu.VMEM((BPW,), jnp.float32),
            pltpu.VMEM((NPAD, H), jnp.float32),
            pltpu.VMEM((BPW, NPAD), jnp.float32),
            pltpu.VMEM((BPW,), jnp.int32),
            pltpu.SemaphoreType.DMA,
        ],
        compiler_params=pltpu.CompilerParams(needs_layout_passes=False,
                                             use_tc_tiling_on_sc=False),
        interpret=interpret,
    )
    def sc_kernel(embed_hbm, atn_hbm, v_hbm, c_hbm, attn_hbm, idxo_hbm,
                  idx_v, v_v, c_v, rows_v, attn_v, idxo_v, sem):
        wid = lax.axis_index("s") * _NC + lax.axis_index("c")
        base = wid * BPW
        pltpu.sync_copy(atn_hbm.at[pl.ds(base, BPW)], idx_v)
        pltpu.sync_copy(v_hbm.at[pl.ds(base, BPW)], v_v)
        pltpu.sync_copy(c_hbm.at[pl.ds(base, BPW)], c_v)
        lanes = lax.iota(jnp.int32, _L)
        neg_inf = jnp.float32(-jnp.inf)

        def per_row(b, carry):
            pltpu.async_copy(
                embed_hbm.at[idx_v.at[b]], rows_v.at[pl.ds(0, N)], sem).wait()
            c_chunk = c_v[pl.ds((b // _L) * _L, _L)]
            c_bcast = _vbcast(c_chunk, b % _L)
            vch = [v_v[b, pl.ds(hc * _L, _L)] for hc in range(HCH)]
            for nc in range(NCH):
                acc = c_bcast
                n_idx = lanes + (nc * _L)
                for hc in range(HCH):
                    for hh in range(_L):
                        h = hc * _L + hh
                        t = plsc.load_gather(
                            rows_v, [n_idx, jnp.full((_L,), h, jnp.int32)])
                        acc = acc + t * _vbcast(vch[hc], hh)
                if (nc + 1) * _L > N:
                    acc = jnp.where(n_idx < N, acc, neg_inf)
                attn_v[b, pl.ds(nc * _L, _L)] = acc
            return carry

        lax.fori_loop(0, BPW, per_row, 0)

        for bc in range(BPW // _L):
            b_idx = lanes + bc * _L
            curmax = jnp.full((_L,), neg_inf, jnp.float32)
            curarg = jnp.full((_L,), 0, jnp.int32)
            for n in range(N):
                t = plsc.load_gather(
                    attn_v, [b_idx, jnp.full((_L,), n, jnp.int32)])
                cond = t > curmax
                curmax = jnp.where(cond, t, curmax)
                curarg = jnp.where(cond, jnp.full((_L,), n, jnp.int32), curarg)
            idxo_v[pl.ds(bc * _L, _L)] = curarg

        pltpu.sync_copy(attn_v, attn_hbm.at[pl.ds(base, BPW)])
        pltpu.sync_copy(idxo_v, idxo_hbm.at[pl.ds(base, BPW)])

    return sc_kernel(embed, atn, v, c)


def kernel(stim, atnTensor, embed, W1, b1, Wfc, bfc):
    N = atnTensor.shape[1]
    v, c = _tc_prologue(stim, W1, b1, Wfc, bfc)
    attn_pad, idx = _sc_main(embed, atnTensor, v, c.reshape(-1))
    return attn_pad[:, :N], idx


# speedup vs baseline: 1.0575x; 1.0575x over previous
import functools

import jax
import jax.numpy as jnp
from jax import lax
from jax.experimental import pallas as pl
from jax.experimental.pallas import tpu as pltpu
from jax.experimental.pallas import tpu_sc as plsc

_NC = 2
_NS = 16
_NW = _NC * _NS

_CHUNK = 128
_NBUF = 8
_DEPTH = 4


def _sc_gather(embed, atn_chunked):
    n_chunks_total, chunk = atn_chunked.shape
    H = embed.shape[1]
    R = n_chunks_total * chunk
    cpw = n_chunks_total // _NW

    mesh = plsc.VectorSubcoreMesh(core_axis_name="c", subcore_axis_name="s",
                                  num_cores=_NC, num_subcores=_NS)

    @functools.partial(
        pl.kernel,
        mesh=mesh,
        out_type=jax.ShapeDtypeStruct((R, H), jnp.float32),
        scratch_types=(
            [pltpu.VMEM((cpw, chunk), jnp.int32)]
            + [pltpu.VMEM((chunk, H), jnp.float32) for _ in range(_NBUF)]
            + [pltpu.SemaphoreType.DMA for _ in range(2 * _NBUF)]
        ),
        compiler_params=pltpu.CompilerParams(needs_layout_passes=False,
                                             use_tc_tiling_on_sc=False),
    )
    def gather_kernel(embed_hbm, idx_hbm, out_hbm, idx_v, *rest):
        bufs = rest[:_NBUF]
        gsem = rest[_NBUF:2 * _NBUF]
        ssem = rest[2 * _NBUF:]
        wid = lax.axis_index("s") * _NC + lax.axis_index("c")
        cbase = wid * cpw
        rbase = cbase * chunk
        pltpu.sync_copy(idx_hbm.at[pl.ds(cbase, cpw)], idx_v)

        gd, sd = {}, {}

        def start_gather(c):
            j = c % _NBUF
            gd[c] = pltpu.async_copy(embed_hbm.at[idx_v.at[c]], bufs[j],
                                     gsem[j])

        def start_store(c):
            j = c % _NBUF
            sd[c] = pltpu.async_copy(
                bufs[j], out_hbm.at[pl.ds(rbase + c * chunk, chunk)], ssem[j])

        for c in range(min(_DEPTH, cpw)):
            start_gather(c)
        for s in range(cpw):
            gd.pop(s).wait()
            start_store(s)
            p = s + _DEPTH
            if p < cpw:
                if p >= _NBUF:
                    sd.pop(p - _NBUF).wait()
                start_gather(p)
        for c in sorted(sd):
            sd.pop(c).wait()

    return gather_kernel(embed, atn_chunked)


def _tc_score(targs3, stim, W1, b1, Wfc, bfc):
    B, N, H = targs3.shape
    BLK = 256
    b1r = b1.reshape(1, 1, H)
    wfcr = Wfc.reshape(1, 1, H)
    bfcr = bfc.reshape(1, 1)

    def body(t_ref, stim_ref, w1_ref, b1_ref, wfc_ref, bfc_ref,
             attn_ref, idx_ref):
        t2 = t_ref[...].reshape(BLK * N, H)
        x2 = lax.dot_general(
            t2.astype(jnp.bfloat16), w1_ref[...].astype(jnp.bfloat16),
            (((1,), (0,)), ((), ())), preferred_element_type=jnp.float32)
        x3 = x2.reshape(BLK, N, H) + b1_ref[...]
        y3 = x3 * stim_ref[...].reshape(BLK, 1, H)
        wfc_b = wfc_ref[...].astype(jnp.bfloat16).astype(jnp.float32)
        z3 = y3.astype(jnp.bfloat16).astype(jnp.float32) * wfc_b
        attn = jnp.sum(z3, axis=2) + bfc_ref[...]
        attn_ref[...] = attn
        m = jnp.max(attn, axis=1, keepdims=True)
        n_iota = lax.broadcasted_iota(jnp.int32, (BLK, N), 1)
        cand = jnp.where(attn == m, n_iota, N)
        idx_ref[...] = jnp.min(cand, axis=1, keepdims=True)

    grid = (B // BLK,)
    attn, idx2 = pl.pallas_call(
        body,
        grid=grid,
        in_specs=[
            pl.BlockSpec((BLK, N, H), lambda i: (i, 0, 0)),
            pl.BlockSpec((BLK, H), lambda i: (i, 0)),
            pl.BlockSpec((H, H), lambda i: (0, 0)),
            pl.BlockSpec((1, 1, H), lambda i: (0, 0, 0)),
            pl.BlockSpec((1, 1, H), lambda i: (0, 0, 0)),
            pl.BlockSpec((1, 1), lambda i: (0, 0)),
        ],
        out_specs=[
            pl.BlockSpec((BLK, N), lambda i: (i, 0)),
            pl.BlockSpec((BLK, 1), lambda i: (i, 0)),
        ],
        out_shape=[
            jax.ShapeDtypeStruct((B, N), jnp.float32),
            jax.ShapeDtypeStruct((B, 1), jnp.int32),
        ],
    )(targs3, stim, W1, b1r, wfcr, bfcr)
    return attn, idx2


def kernel(stim, atnTensor, embed, W1, b1, Wfc, bfc):
    B, N = atnTensor.shape
    H = embed.shape[1]
    atn_chunked = atnTensor.reshape(B * N // _CHUNK, _CHUNK)
    targs = _sc_gather(embed, atn_chunked)
    attn, idx2 = _tc_score(targs.reshape(B, N, H), stim, W1, b1, Wfc, bfc)
    return attn, idx2.reshape(B)
